# baseline (speedup 1.0000x reference)
import functools

import jax
import jax.numpy as jnp
from jax import lax
from jax.experimental import pallas as pl
from jax.experimental.pallas import tpu as pltpu
from jax.experimental.pallas import tpu_sc as plsc

QN = 256
EPS = 1e-8
K_ROWS = 100000
D = 1024
BLK = 2000
NB = K_ROWS // BLK
PADW = 2048
PAD_VAL = -2.0

SC_CORES = 2
SC_SUBCORES = 16
SC_WORKERS = SC_CORES * SC_SUBCORES
ROWS_PER_WORKER = QN // SC_WORKERS


def _score_kernel(img_ref, blk_ref, nrm_ref, out_ref):
    img_n = img_ref[...].astype(jnp.bfloat16)
    blk = blk_ref[...]
    txt_n = (blk / nrm_ref[...]).astype(jnp.bfloat16)
    dots = lax.dot_general(img_n, txt_n, (((1,), (1,)), ((), ())),
                           preferred_element_type=jnp.float32)
    scores = jnp.sum(dots, axis=0, keepdims=True) * (1.0 / 32.0)
    pad = jnp.full((1, PADW - BLK), PAD_VAL, jnp.float32)
    out_ref[...] = jnp.concatenate([scores, pad], axis=1).reshape(1, 1, PADW)


def _topk_kernel(s_in_ref, idx_ref, s_ref):
    s_ref[...] = s_in_ref[...]
    rowi = lax.broadcasted_iota(jnp.int32, (NB, PADW), 0)
    coli = lax.broadcasted_iota(jnp.int32, (NB, PADW), 1)
    flat = rowi * BLK + coli
    idx_ref[...] = jnp.zeros((QN,), jnp.int32)
    out_iota = lax.iota(jnp.int32, QN)

    def body(r, _):
        s = s_ref[...]
        m = jnp.max(s)
        cand = jnp.where(s == m, flat, jnp.int32(2**31 - 1))
        fi = jnp.min(cand)
        idx_ref[...] = jnp.where(out_iota == r, fi, idx_ref[...])
        s_ref[...] = jnp.where(flat == fi, PAD_VAL, s)
        return 0

    lax.fori_loop(0, QN, body, 0)


def _gather_body(text_hbm, idx_hbm, out_hbm, idx_v, rows_v, sem):
    wid = lax.axis_index("s") * SC_CORES + lax.axis_index("c")
    base = wid * ROWS_PER_WORKER
    pltpu.sync_copy(idx_hbm.at[pl.ds(base, ROWS_PER_WORKER)], idx_v)
    pltpu.async_copy(text_hbm.at[idx_v], rows_v, sem).wait()
    pltpu.sync_copy(rows_v, out_hbm.at[pl.ds(base, ROWS_PER_WORKER)])


def kernel(image_features, text_embedding):
    img_n = image_features / jnp.maximum(
        jnp.linalg.norm(image_features, axis=-1, keepdims=True), EPS)
    txt_norm = jnp.maximum(
        jnp.linalg.norm(text_embedding, axis=-1, keepdims=True), EPS)

    scores = pl.pallas_call(
        _score_kernel,
        grid=(NB,),
        in_specs=[
            pl.BlockSpec((32, D), lambda i: (0, 0)),
            pl.BlockSpec((BLK, D), lambda i: (i, 0)),
            pl.BlockSpec((BLK, 1), lambda i: (i, 0)),
        ],
        out_specs=pl.BlockSpec((1, 1, PADW), lambda i: (i, 0, 0)),
        out_shape=jax.ShapeDtypeStruct((NB, 1, PADW), jnp.float32),
    )(img_n, text_embedding, txt_norm)
    scores = scores.reshape(NB, PADW)

    indices = pl.pallas_call(
        _topk_kernel,
        in_specs=[pl.BlockSpec(memory_space=pltpu.VMEM)],
        out_specs=pl.BlockSpec(memory_space=pltpu.VMEM),
        out_shape=jax.ShapeDtypeStruct((QN,), jnp.int32),
        scratch_shapes=[pltpu.VMEM((NB, PADW), jnp.float32)],
    )(scores)

    gather = functools.partial(
        pl.kernel,
        mesh=plsc.VectorSubcoreMesh(core_axis_name="c", subcore_axis_name="s"),
        out_type=jax.ShapeDtypeStruct((QN, D), jnp.float32),
        scratch_types=[
            pltpu.VMEM((ROWS_PER_WORKER,), jnp.int32),
            pltpu.VMEM((ROWS_PER_WORKER, D), jnp.float32),
            pltpu.SemaphoreType.DMA,
        ],
    )(_gather_body)
    return gather(text_embedding, indices)


# speedup vs baseline: 1.0735x; 1.0735x over previous
import functools

import jax
import jax.numpy as jnp
from jax import lax
from jax.experimental import pallas as pl
from jax.experimental.pallas import tpu as pltpu
from jax.experimental.pallas import tpu_sc as plsc

QN = 256
EPS = 1e-8
K_ROWS = 100000
D = 1024
BLK = 2000
NB = K_ROWS // BLK
PADW = 2048
PAD_VAL = -2.0

SC_CORES = 2
SC_SUBCORES = 16
SC_WORKERS = SC_CORES * SC_SUBCORES
ROWS_PER_WORKER = QN // SC_WORKERS


def _score_kernel(img_ref, blk_ref, nrm_ref, out_ref):
    img_n = img_ref[...].astype(jnp.bfloat16)
    blk = blk_ref[...]
    txt_n = (blk / nrm_ref[...]).astype(jnp.bfloat16)
    dots = lax.dot_general(img_n, txt_n, (((1,), (1,)), ((), ())),
                           preferred_element_type=jnp.float32)
    scores = jnp.sum(dots, axis=0, keepdims=True) * (1.0 / 32.0)
    pad = jnp.full((1, PADW - BLK), PAD_VAL, jnp.float32)
    out_ref[...] = jnp.concatenate([scores, pad], axis=1).reshape(1, 1, PADW)


def _topk_kernel(s_in_ref, idx_ref, s_ref):
    s_ref[...] = s_in_ref[...]
    rowmax = jnp.max(s_in_ref[...], axis=1, keepdims=True)
    rowiota = lax.broadcasted_iota(jnp.int32, (NB, 1), 0)
    coliota = lax.broadcasted_iota(jnp.int32, (1, PADW), 1)
    idx_ref[...] = jnp.zeros((QN,), jnp.int32)
    out_iota = lax.iota(jnp.int32, QN)

    def body(t, rowmax):
        m = jnp.max(rowmax)
        r = jnp.min(jnp.where(rowmax == m, rowiota, NB))
        srow = s_ref[pl.ds(r, 1), :]
        c = jnp.min(jnp.where(srow == m, coliota, PADW))
        idx_ref[...] = jnp.where(out_iota == t, r * BLK + c, idx_ref[...])
        new_row = jnp.where(coliota == c, PAD_VAL, srow)
        s_ref[pl.ds(r, 1), :] = new_row
        return jnp.where(rowiota == r, jnp.max(new_row), rowmax)

    lax.fori_loop(0, QN, body, rowmax)


def _gather_body(text_hbm, idx_hbm, out_hbm, idx_v, rows_v, sem):
    wid = lax.axis_index("s") * SC_CORES + lax.axis_index("c")
    base = wid * ROWS_PER_WORKER
    pltpu.sync_copy(idx_hbm.at[pl.ds(base, ROWS_PER_WORKER)], idx_v)
    pltpu.async_copy(text_hbm.at[idx_v], rows_v, sem).wait()
    pltpu.sync_copy(rows_v, out_hbm.at[pl.ds(base, ROWS_PER_WORKER)])


def kernel(image_features, text_embedding):
    img_n = image_features / jnp.maximum(
        jnp.linalg.norm(image_features, axis=-1, keepdims=True), EPS)
    txt_norm = jnp.maximum(
        jnp.linalg.norm(text_embedding, axis=-1, keepdims=True), EPS)

    scores = pl.pallas_call(
        _score_kernel,
        grid=(NB,),
        in_specs=[
            pl.BlockSpec((32, D), lambda i: (0, 0)),
            pl.BlockSpec((BLK, D), lambda i: (i, 0)),
            pl.BlockSpec((BLK, 1), lambda i: (i, 0)),
        ],
        out_specs=pl.BlockSpec((1, 1, PADW), lambda i: (i, 0, 0)),
        out_shape=jax.ShapeDtypeStruct((NB, 1, PADW), jnp.float32),
    )(img_n, text_embedding, txt_norm)
    scores = scores.reshape(NB, PADW)

    indices = pl.pallas_call(
        _topk_kernel,
        in_specs=[pl.BlockSpec(memory_space=pltpu.VMEM)],
        out_specs=pl.BlockSpec(memory_space=pltpu.VMEM),
        out_shape=jax.ShapeDtypeStruct((QN,), jnp.int32),
        scratch_shapes=[pltpu.VMEM((NB, PADW), jnp.float32)],
    )(scores)

    gather = functools.partial(
        pl.kernel,
        mesh=plsc.VectorSubcoreMesh(core_axis_name="c", subcore_axis_name="s"),
        out_type=jax.ShapeDtypeStruct((QN, D), jnp.float32),
        scratch_types=[
            pltpu.VMEM((ROWS_PER_WORKER,), jnp.int32),
            pltpu.VMEM((ROWS_PER_WORKER, D), jnp.float32),
            pltpu.SemaphoreType.DMA,
        ],
    )(_gather_body)
    return gather(text_embedding, indices)
